# baseline (speedup 1.0000x reference)
import functools

import jax
import jax.numpy as jnp
from jax import lax
from jax.experimental import pallas as pl
from jax.experimental.pallas import tpu as pltpu
from jax.experimental.pallas import tpu_sc as plsc

B = 4096
F = 26
ROWS = 100001
D = 64

_INFO = plsc.get_sparse_core_info()
NC = _INFO.num_cores
NS = _INFO.num_subcores
L = _INFO.num_lanes
NW = NC * NS

TOT = B * F
PER_W = TOT // NW
CHUNK = 128
NCHUNK = PER_W // CHUNK
VREGS = PER_W // L

_mesh = plsc.VectorSubcoreMesh(core_axis_name="c", subcore_axis_name="s")


@functools.partial(
    pl.kernel,
    mesh=_mesh,
    out_type=jax.ShapeDtypeStruct((TOT, D), jnp.float32),
    compiler_params=pltpu.CompilerParams(use_tc_tiling_on_sc=False),
    scratch_types=[
        pltpu.VMEM((PER_W,), jnp.int32),
        pltpu.VMEM((PER_W,), jnp.int32),
        pltpu.VMEM((CHUNK, D), jnp.float32),
        pltpu.SemaphoreType.DMA,
    ],
)
def _emb_lookup(idx_hbm, tab_hbm, out_hbm, idx_v, flat_v, rows_v, sem):
    wid = lax.axis_index("s") * NC + lax.axis_index("c")
    base = wid * PER_W

    pltpu.sync_copy(idx_hbm.at[pl.ds(base, PER_W)], idx_v)

    def idx_body(k, _):
        pos = base + k * L + lax.iota(jnp.int32, 16)
        field = lax.rem(pos, F)
        flat_v[pl.ds(k * L, L)] = idx_v[pl.ds(k * L, L)] + field * ROWS
        return 0

    lax.fori_loop(0, VREGS, idx_body, 0)

    def gather_body(j, _):
        off = j * CHUNK
        pltpu.async_copy(
            tab_hbm.at[flat_v.at[pl.ds(off, CHUNK)]], rows_v, sem
        ).wait()
        pltpu.sync_copy(rows_v, out_hbm.at[pl.ds(base + off, CHUNK)])
        return 0

    lax.fori_loop(0, NCHUNK, gather_body, 0)


def kernel(indices, tables):
    idx_flat = indices.reshape(TOT).astype(jnp.int32)
    tab_flat = tables.reshape(F * ROWS, D)
    out = _emb_lookup(idx_flat, tab_flat)
    return out.reshape(B, F, D)


# speedup vs baseline: 20.3920x; 20.3920x over previous
import functools

import jax
import jax.numpy as jnp
from jax import lax
from jax.experimental import pallas as pl
from jax.experimental.pallas import tpu as pltpu
from jax.experimental.pallas import tpu_sc as plsc

B = 4096
F = 26
ROWS = 100001
D = 64

_INFO = plsc.get_sparse_core_info()
NC = _INFO.num_cores
NS = _INFO.num_subcores
L = _INFO.num_lanes
NW = NC * NS

G = F * D
PER_W = G // NW
BVREGS = B // L

_mesh = plsc.VectorSubcoreMesh(core_axis_name="c", subcore_axis_name="s")


@functools.partial(
    pl.kernel,
    mesh=_mesh,
    out_type=jax.ShapeDtypeStruct((G, B), jnp.float32),
    compiler_params=pltpu.CompilerParams(needs_layout_passes=False),
    scratch_types=[
        pltpu.VMEM((ROWS,), jnp.float32),
        pltpu.VMEM((B,), jnp.int32),
        pltpu.VMEM((B,), jnp.float32),
    ],
)
def _emb_lookup(idx_hbm, tab_hbm, out_hbm, row_v, idx_v, res_v):
    wid = lax.axis_index("s") * NC + lax.axis_index("c")
    base = wid * PER_W

    def row_body(t, _):
        g = base + t
        f = g // D
        pltpu.sync_copy(idx_hbm.at[f], idx_v)
        pltpu.sync_copy(tab_hbm.at[g], row_v)

        def gather_body(k, _):
            iv = idx_v[pl.ds(k * L, L)]
            res_v[pl.ds(k * L, L)] = plsc.load_gather(row_v, [iv])
            return 0

        lax.fori_loop(0, BVREGS, gather_body, 0)
        pltpu.sync_copy(res_v, out_hbm.at[g])
        return 0

    lax.fori_loop(0, PER_W, row_body, 0)


def kernel(indices, tables):
    idx_t = indices.astype(jnp.int32).T
    tab_soa = tables.transpose(0, 2, 1).reshape(G, ROWS)
    out_soa = _emb_lookup(idx_t, tab_soa)
    return out_soa.reshape(F, D, B).transpose(2, 0, 1)


# speedup vs baseline: 20.8776x; 1.0238x over previous
import functools

import jax
import jax.numpy as jnp
from jax import lax
from jax.experimental import pallas as pl
from jax.experimental.pallas import tpu as pltpu
from jax.experimental.pallas import tpu_sc as plsc

B = 4096
F = 26
ROWS = 100001
D = 64

_INFO = plsc.get_sparse_core_info()
NC = _INFO.num_cores
NS = _INFO.num_subcores
L = _INFO.num_lanes
NW = NC * NS

G = F * D
PER_W = G // NW
RB = 4
UNROLL = 16

_mesh = plsc.VectorSubcoreMesh(core_axis_name="c", subcore_axis_name="s")


@functools.partial(
    pl.kernel,
    mesh=_mesh,
    out_type=jax.ShapeDtypeStruct((G, B), jnp.float32),
    compiler_params=pltpu.CompilerParams(needs_layout_passes=False),
    scratch_types=[
        pltpu.VMEM((ROWS,), jnp.float32),
        pltpu.VMEM((2, B), jnp.int32),
        pltpu.VMEM((RB, B), jnp.float32),
    ],
)
def _emb_lookup(idx_hbm, tab_hbm, out_hbm, row_v, idx_v, res_v):
    wid = lax.axis_index("s") * NC + lax.axis_index("c")
    gbase = wid * PER_W
    f0 = gbase // D
    pltpu.sync_copy(idx_hbm.at[f0], idx_v.at[0])
    pltpu.sync_copy(idx_hbm.at[jnp.minimum(f0 + 1, F - 1)], idx_v.at[1])

    def super_body(t, _):
        g0 = gbase + t * RB
        for u in range(RB):
            g = g0 + u
            pltpu.sync_copy(tab_hbm.at[g], row_v)
            frel = g // D - f0

            def gather_body(i, _):
                for v in range(UNROLL):
                    s = pl.ds((i * UNROLL + v) * L, L)
                    iv = idx_v[frel, s]
                    res_v[u, s] = plsc.load_gather(row_v, [iv])
                return 0

            lax.fori_loop(0, B // (L * UNROLL), gather_body, 0)

        pltpu.sync_copy(res_v, out_hbm.at[pl.ds(g0, RB)])
        return 0

    lax.fori_loop(0, PER_W // RB, super_body, 0)


def kernel(indices, tables):
    idx_t = indices.astype(jnp.int32).T
    tab_soa = tables.transpose(0, 2, 1).reshape(G, ROWS)
    out_soa = _emb_lookup(idx_t, tab_soa)
    return out_soa.reshape(F, D, B).transpose(2, 0, 1)


# speedup vs baseline: 20.8838x; 1.0003x over previous
import functools

import jax
import jax.numpy as jnp
from jax import lax
from jax.experimental import pallas as pl
from jax.experimental.pallas import tpu as pltpu
from jax.experimental.pallas import tpu_sc as plsc

B = 4096
F = 26
ROWS = 100001
D = 64

_INFO = plsc.get_sparse_core_info()
NC = _INFO.num_cores
NS = _INFO.num_subcores
L = _INFO.num_lanes
NW = NC * NS

G = F * D
PER_W = G // NW
RB = 4
UNROLL = 16
NQ = 4
_Q = ((ROWS // NQ) // 128) * 128 + 128
QOFF = [q * _Q for q in range(NQ)]
QLEN = [_Q] * (NQ - 1) + [ROWS - (NQ - 1) * _Q]

_mesh = plsc.VectorSubcoreMesh(core_axis_name="c", subcore_axis_name="s")


@functools.partial(
    pl.kernel,
    mesh=_mesh,
    out_type=jax.ShapeDtypeStruct((G, B), jnp.float32),
    compiler_params=pltpu.CompilerParams(needs_layout_passes=False),
    scratch_types=[
        pltpu.VMEM((ROWS,), jnp.float32),
        pltpu.VMEM((2, B), jnp.int32),
        pltpu.VMEM((RB, B), jnp.float32),
    ],
)
def _emb_lookup(idx_hbm, tab_hbm, out_hbm, row_v, idx_v, res_v):
    wid = lax.axis_index("s") * NC + lax.axis_index("c")
    gbase = wid * PER_W
    f0 = gbase // D
    pltpu.sync_copy(idx_hbm.at[f0], idx_v.at[0])
    pltpu.sync_copy(idx_hbm.at[jnp.minimum(f0 + 1, F - 1)], idx_v.at[1])

    def super_body(t, _):
        g0 = gbase + t * RB
        for u in range(RB):
            g = g0 + u
            pltpu.sync_copy(tab_hbm.at[g], row_v)
            frel = g // D - f0

            def gather_body(i, _):
                for v in range(UNROLL):
                    s = pl.ds((i * UNROLL + v) * L, L)
                    iv = idx_v[frel, s]
                    res_v[u, s] = plsc.load_gather(row_v, [iv])
                return 0

            lax.fori_loop(0, B // (L * UNROLL), gather_body, 0)

        pltpu.sync_copy(res_v, out_hbm.at[pl.ds(g0, RB)])
        return 0

    lax.fori_loop(0, PER_W // RB, super_body, 0)


def kernel(indices, tables):
    idx_t = indices.astype(jnp.int32).T
    tab_soa = tables.transpose(0, 2, 1).reshape(G, ROWS)
    out_soa = _emb_lookup(idx_t, tab_soa)
    return out_soa.reshape(F, D, B).transpose(2, 0, 1)
